# baseline (speedup 1.0000x reference)
import functools

import jax
import jax.numpy as jnp
from jax import lax
from jax.experimental import pallas as pl
from jax.experimental.pallas import tpu as pltpu
from jax.experimental.pallas import tpu_sc as plsc

N = 100000
E = 3200000
G = 64
INCH = 128
HID = 12
HP = 16
OUT = 16
NRD = 102400
NR = 100016
CH = 512
RPC = CH // 128
NW = 32
NCH = -(-E // (NW * CH))
NCH2 = NCH // 2
EP = NW * NCH * CH
RPTD = NRD // 16
RPT = NR // 16
RB = 5000
NB = N // RB

_mesh = plsc.VectorSubcoreMesh(core_axis_name="c", subcore_axis_name="s")


@functools.partial(
    pl.kernel,
    out_type=jax.ShapeDtypeStruct((2, NRD), jnp.float32),
    mesh=_mesh,
    scratch_types=[
        pltpu.VMEM((128,), jnp.float32),
        pltpu.VMEM((1024,), jnp.float32),
        pltpu.VMEM((RPC, 128), jnp.int32),
        pltpu.VMEM_SHARED((NRD,), jnp.float32),
        pltpu.SemaphoreType.DMA,
    ],
    compiler_params=pltpu.CompilerParams(use_tc_tiling_on_sc=False),
)
def _deg_kernel(dst_hbm, deg_out, ones_v, zeros_v, idx_v, acc, sem):
    cid = lax.axis_index("c")
    sid = lax.axis_index("s")
    wid = cid * 16 + sid
    for k in range(8):
        ones_v[pl.ds(k * 16, 16)] = jnp.ones((16,), jnp.float32)

    def _z(i, c):
        zeros_v[pl.ds(i * 16, 16)] = jnp.zeros((16,), jnp.float32)
        return c

    lax.fori_loop(0, 64, _z, 0)
    base = sid * RPTD
    for k in range(6):
        pltpu.sync_copy(zeros_v, acc.at[pl.ds(base + k * 1024, 1024)])
    pltpu.sync_copy(zeros_v.at[pl.ds(0, 256)], acc.at[pl.ds(base + 6144, 256)])
    plsc.subcore_barrier()

    def _chunk(c, carry):
        row0 = (wid * NCH + c) * RPC
        pltpu.sync_copy(dst_hbm.at[pl.ds(row0, RPC)], idx_v)
        cps = [
            pltpu.async_copy(ones_v, acc.at[idx_v.at[j]], sem, add=True)
            for j in range(RPC)
        ]
        for cp in cps:
            cp.wait()
        return carry

    lax.fori_loop(0, NCH, _chunk, 0)
    plsc.subcore_barrier()
    pltpu.sync_copy(acc.at[pl.ds(base, RPTD)],
                    deg_out.at[cid, pl.ds(base, RPTD)])


@functools.partial(
    pl.kernel,
    out_type=jax.ShapeDtypeStruct((2, NR, HP), jnp.float32),
    mesh=_mesh,
    scratch_types=[
        pltpu.VMEM((RPC, 128), jnp.int32),
        pltpu.VMEM((RPC, 128), jnp.int32),
        pltpu.VMEM((CH, HP), jnp.float32),
        pltpu.VMEM((RPC, 128), jnp.int32),
        pltpu.VMEM((RPC, 128), jnp.int32),
        pltpu.VMEM((CH, HP), jnp.float32),
        pltpu.VMEM_SHARED((NR, HP), jnp.float32),
        pltpu.SemaphoreType.DMA,
        pltpu.SemaphoreType.DMA,
        pltpu.SemaphoreType.DMA,
        pltpu.SemaphoreType.DMA,
    ],
    compiler_params=pltpu.CompilerParams(use_tc_tiling_on_sc=False),
)
def _prop_kernel(src_hbm, dst_hbm, tab_hbm, zer_hbm, out_hbm,
                 isrcA, idstA, rowsA, isrcB, idstB, rowsB, acc,
                 gsemA, ssemA, gsemB, ssemB):
    cid = lax.axis_index("c")
    sid = lax.axis_index("s")
    wid = cid * 16 + sid
    base = sid * RPT
    pltpu.sync_copy(zer_hbm, acc.at[pl.ds(base, RPT)])
    plsc.subcore_barrier()

    def _load(c, isrc, idst):
        row0 = (wid * NCH + c) * RPC
        pltpu.sync_copy(src_hbm.at[pl.ds(row0, RPC)], isrc)
        pltpu.sync_copy(dst_hbm.at[pl.ds(row0, RPC)], idst)

    def _fire_gather(isrc, rows, sem):
        for j in range(RPC):
            pltpu.async_copy(tab_hbm.at[isrc.at[j]],
                             rows.at[pl.ds(j * 128, 128)], sem)

    def _fire_scatter(idst, rows, sem):
        for j in range(RPC):
            pltpu.async_copy(rows.at[pl.ds(j * 128, 128)],
                             acc.at[idst.at[j]], sem, add=True)

    def _drain(rows, sem):
        pltpu.make_async_copy(zer_hbm.at[pl.ds(0, CH)], rows, sem).wait()

    _load(0, isrcA, idstA)
    _fire_gather(isrcA, rowsA, gsemA)

    def _pair(i, carry):
        a = 2 * i
        b = a + 1
        _drain(rowsA, gsemA)
        _fire_scatter(idstA, rowsA, ssemA)

        @pl.when(i > 0)
        def _():
            _drain(rowsB, ssemB)

        _load(b, isrcB, idstB)
        _fire_gather(isrcB, rowsB, gsemB)
        _drain(rowsB, gsemB)
        _fire_scatter(idstB, rowsB, ssemB)
        _drain(rowsA, ssemA)

        @pl.when(i < NCH2 - 1)
        def _():
            _load(a + 2, isrcA, idstA)
            _fire_gather(isrcA, rowsA, gsemA)

        return carry

    lax.fori_loop(0, NCH2, _pair, 0)
    _drain(rowsB, ssemB)
    plsc.subcore_barrier()
    pltpu.sync_copy(acc.at[pl.ds(base, RPT)],
                    out_hbm.at[cid, pl.ds(base, RPT)])


def _mm1_body(x_ref, deg_ref, w_ref, p_ref):
    d = deg_ref[0, 0] + deg_ref[1, 0] + 1.0
    dinv = lax.rsqrt(d)
    h = jnp.dot(x_ref[...], w_ref[...], preferred_element_type=jnp.float32)
    p_ref[...] = h * dinv


_mm1 = pl.pallas_call(
    _mm1_body,
    grid=(NB,),
    in_specs=[
        pl.BlockSpec((RB, INCH), lambda i: (i, 0)),
        pl.BlockSpec((2, 1, RB, 1), lambda i: (0, i, 0, 0)),
        pl.BlockSpec((INCH, HP), lambda i: (0, 0)),
    ],
    out_specs=pl.BlockSpec((RB, HP), lambda i: (i, 0)),
    out_shape=jax.ShapeDtypeStruct((NR, HP), jnp.float32),
)


def _mid_body(s_ref, p_ref, deg_ref, b1_ref, w2_ref, o_ref):
    d = deg_ref[0, 0] + deg_ref[1, 0] + 1.0
    dinv = lax.rsqrt(d)
    h = (s_ref[0] + s_ref[1] + p_ref[...]) * dinv + b1_ref[...]
    a = jnp.where(h >= 0.0, h, 0.01 * h)
    o_ref[...] = jnp.dot(a, w2_ref[...],
                         preferred_element_type=jnp.float32) * dinv


_mid = pl.pallas_call(
    _mid_body,
    grid=(NB,),
    in_specs=[
        pl.BlockSpec((2, RB, HP), lambda i: (0, i, 0)),
        pl.BlockSpec((RB, HP), lambda i: (i, 0)),
        pl.BlockSpec((2, 1, RB, 1), lambda i: (0, i, 0, 0)),
        pl.BlockSpec((1, HP), lambda i: (0, 0)),
        pl.BlockSpec((HP, HP), lambda i: (0, 0)),
    ],
    out_specs=pl.BlockSpec((RB, HP), lambda i: (i, 0)),
    out_shape=jax.ShapeDtypeStruct((NR, HP), jnp.float32),
)


def _pool_body(s_ref, p_ref, deg_ref, b2_ref, batch_ref, o_ref):
    i = pl.program_id(0)

    @pl.when(i == 0)
    def _():
        o_ref[...] = jnp.zeros_like(o_ref)

    d = deg_ref[0, 0] + deg_ref[1, 0] + 1.0
    dinv = lax.rsqrt(d)
    h = (s_ref[0] + s_ref[1] + p_ref[...]) * dinv + b2_ref[...]
    a = jnp.where(h >= 0.0, h, 0.01 * h)
    seg = batch_ref[0]
    ids = lax.broadcasted_iota(jnp.int32, (G, RB), 0)
    onehot = (seg == ids).astype(jnp.float32)
    o_ref[...] += jnp.dot(onehot, a, preferred_element_type=jnp.float32)


_pool = pl.pallas_call(
    _pool_body,
    grid=(NB,),
    in_specs=[
        pl.BlockSpec((2, RB, HP), lambda i: (0, i, 0)),
        pl.BlockSpec((RB, HP), lambda i: (i, 0)),
        pl.BlockSpec((2, 1, RB, 1), lambda i: (0, i, 0, 0)),
        pl.BlockSpec((1, HP), lambda i: (0, 0)),
        pl.BlockSpec((1, 1, RB), lambda i: (i, 0, 0)),
    ],
    out_specs=pl.BlockSpec((G, HP), lambda i: (0, 0)),
    out_shape=jax.ShapeDtypeStruct((G, HP), jnp.float32),
)


def _head_body(pool_ref, ws_ref, w0_ref, b0_ref, o_ref):
    o_ref[...] = (jnp.dot(pool_ref[...], ws_ref[...],
                          preferred_element_type=jnp.float32)
                  + w0_ref[...] + b0_ref[...])


_head = pl.pallas_call(
    _head_body,
    grid=(1,),
    in_specs=[
        pl.BlockSpec((G, HP), lambda i: (0, 0)),
        pl.BlockSpec((HP, OUT), lambda i: (0, 0)),
        pl.BlockSpec((1, OUT), lambda i: (0, 0)),
        pl.BlockSpec((1, OUT), lambda i: (0, 0)),
    ],
    out_specs=pl.BlockSpec((G, OUT), lambda i: (0, 0)),
    out_shape=jax.ShapeDtypeStruct((G, OUT), jnp.float32),
)


def kernel(x, edge_index, batch, features,
           W1, b1, W2, b2, Wa, ba, Wb, bb, Wc, bc, Wo, bo):
    del features, Wa, ba, Wb, bb, Wc, bc
    padv = (jnp.arange(EP - E, dtype=jnp.int32) % (NR - N)) + N
    src2 = jnp.concatenate([edge_index[0], padv]).reshape(EP // 128, 128)
    dst2 = jnp.concatenate([edge_index[1], padv]).reshape(EP // 128, 128)
    W1p = jnp.pad(W1, ((0, 0), (0, HP - HID)))
    W2p = jnp.pad(W2, ((0, HP - HID), (0, HP - HID)))
    b1p = jnp.pad(b1, (0, HP - HID)).reshape(1, HP)
    b2p = jnp.pad(b2, (0, HP - HID)).reshape(1, HP)
    Wsp = jnp.pad(Wo[1:, :], ((0, HP - HID), (0, 0)))
    w0 = Wo[0, :].reshape(1, OUT)
    b0 = bo.reshape(1, OUT)
    batch3 = batch.reshape(NB, 1, RB)
    zer = jnp.zeros((RPT, HP), jnp.float32)

    degp = _deg_kernel(dst2)
    deg4 = degp[:, :N].reshape(2, NB, RB, 1)
    p1 = _mm1(x, deg4, W1p)
    s1 = _prop_kernel(src2, dst2, p1, zer)
    p2 = _mid(s1, p1, deg4, b1p, W2p)
    s2 = _prop_kernel(src2, dst2, p2, zer)
    pooled = _pool(s2, p2, deg4, b2p, batch3)
    return _head(pooled, Wsp, w0, b0)


# speedup vs baseline: 1.2239x; 1.2239x over previous
import functools

import jax
import jax.numpy as jnp
from jax import lax
from jax.experimental import pallas as pl
from jax.experimental.pallas import tpu as pltpu
from jax.experimental.pallas import tpu_sc as plsc

N = 100000
E = 3200000
G = 64
INCH = 128
HID = 12
HP = 16
OUT = 16
NRD = 102400
NR = 100016
CH = 1024
RPC = CH // 128
NW = 32
NCH = -(-E // (NW * CH))
EP = NW * NCH * CH
RPTD = NRD // 16
RPT = NR // 16
RB = 5000
NB = N // RB

_mesh = plsc.VectorSubcoreMesh(core_axis_name="c", subcore_axis_name="s")


@functools.partial(
    pl.kernel,
    out_type=jax.ShapeDtypeStruct((2, NRD), jnp.float32),
    mesh=_mesh,
    scratch_types=[
        pltpu.VMEM((128,), jnp.float32),
        pltpu.VMEM((1024,), jnp.float32),
        pltpu.VMEM((RPC, 128), jnp.int32),
        pltpu.VMEM_SHARED((NRD,), jnp.float32),
        pltpu.SemaphoreType.DMA,
    ],
    compiler_params=pltpu.CompilerParams(use_tc_tiling_on_sc=False),
)
def _deg_kernel(dst_hbm, deg_out, ones_v, zeros_v, idx_v, acc, sem):
    cid = lax.axis_index("c")
    sid = lax.axis_index("s")
    wid = cid * 16 + sid
    for k in range(8):
        ones_v[pl.ds(k * 16, 16)] = jnp.ones((16,), jnp.float32)

    def _z(i, c):
        zeros_v[pl.ds(i * 16, 16)] = jnp.zeros((16,), jnp.float32)
        return c

    lax.fori_loop(0, 64, _z, 0)
    base = sid * RPTD
    for k in range(6):
        pltpu.sync_copy(zeros_v, acc.at[pl.ds(base + k * 1024, 1024)])
    pltpu.sync_copy(zeros_v.at[pl.ds(0, 256)], acc.at[pl.ds(base + 6144, 256)])
    plsc.subcore_barrier()

    def _chunk(c, carry):
        row0 = (wid * NCH + c) * RPC
        pltpu.sync_copy(dst_hbm.at[pl.ds(row0, RPC)], idx_v)
        cps = [
            pltpu.async_copy(ones_v, acc.at[idx_v.at[j]], sem, add=True)
            for j in range(RPC)
        ]
        for cp in cps:
            cp.wait()
        return carry

    lax.fori_loop(0, NCH, _chunk, 0)
    plsc.subcore_barrier()
    pltpu.sync_copy(acc.at[pl.ds(base, RPTD)],
                    deg_out.at[cid, pl.ds(base, RPTD)])


@functools.partial(
    pl.kernel,
    out_type=jax.ShapeDtypeStruct((2, NR, HP), jnp.float32),
    mesh=_mesh,
    scratch_types=[
        pltpu.VMEM((RPC, 128), jnp.int32),
        pltpu.VMEM((RPC, 128), jnp.int32),
        pltpu.VMEM((CH, HP), jnp.float32),
        pltpu.VMEM_SHARED((NR, HP), jnp.float32),
        pltpu.SemaphoreType.DMA,
        pltpu.SemaphoreType.DMA,
    ],
    compiler_params=pltpu.CompilerParams(use_tc_tiling_on_sc=False),
)
def _prop_kernel(src_hbm, dst_hbm, tab_hbm, zer_hbm, out_hbm,
                 isrc, idst, rows, acc, gsem, ssem):
    cid = lax.axis_index("c")
    sid = lax.axis_index("s")
    wid = cid * 16 + sid
    base = sid * RPT
    pltpu.sync_copy(zer_hbm, acc.at[pl.ds(base, RPT)])
    plsc.subcore_barrier()

    dummy = jnp.broadcast_to(N + (wid % (NR - N)), (16,)).astype(jnp.int32)
    for k in range(8 * RPC):
        idst[k // 8, pl.ds((k % 8) * 16, 16)] = dummy
    for j in range(RPC):
        pltpu.async_copy(rows.at[pl.ds(j * 128, 128)],
                         acc.at[idst.at[j]], ssem, add=True)

    def _chunk(c, carry):
        pltpu.make_async_copy(zer_hbm.at[pl.ds(0, CH)], rows, ssem).wait()
        row0 = (wid * NCH + c) * RPC
        pltpu.sync_copy(src_hbm.at[pl.ds(row0, RPC)], isrc)
        pltpu.sync_copy(dst_hbm.at[pl.ds(row0, RPC)], idst)
        gs = [
            pltpu.async_copy(tab_hbm.at[isrc.at[j]],
                             rows.at[pl.ds(j * 128, 128)], gsem)
            for j in range(RPC)
        ]
        for j in range(RPC):
            gs[j].wait()
            pltpu.async_copy(rows.at[pl.ds(j * 128, 128)],
                             acc.at[idst.at[j]], ssem, add=True)
        return carry

    lax.fori_loop(0, NCH, _chunk, 0)
    pltpu.make_async_copy(zer_hbm.at[pl.ds(0, CH)], rows, ssem).wait()
    plsc.subcore_barrier()
    pltpu.sync_copy(acc.at[pl.ds(base, RPT)],
                    out_hbm.at[cid, pl.ds(base, RPT)])


def _mm1_body(x_ref, deg_ref, w_ref, p_ref):
    d = deg_ref[0, 0] + deg_ref[1, 0] + 1.0
    dinv = lax.rsqrt(d)
    h = jnp.dot(x_ref[...], w_ref[...], preferred_element_type=jnp.float32)
    p_ref[...] = h * dinv


_mm1 = pl.pallas_call(
    _mm1_body,
    grid=(NB,),
    in_specs=[
        pl.BlockSpec((RB, INCH), lambda i: (i, 0)),
        pl.BlockSpec((2, 1, RB, 1), lambda i: (0, i, 0, 0)),
        pl.BlockSpec((INCH, HP), lambda i: (0, 0)),
    ],
    out_specs=pl.BlockSpec((RB, HP), lambda i: (i, 0)),
    out_shape=jax.ShapeDtypeStruct((NR, HP), jnp.float32),
)


def _mid_body(s_ref, p_ref, deg_ref, b1_ref, w2_ref, o_ref):
    d = deg_ref[0, 0] + deg_ref[1, 0] + 1.0
    dinv = lax.rsqrt(d)
    h = (s_ref[0] + s_ref[1] + p_ref[...]) * dinv + b1_ref[...]
    a = jnp.where(h >= 0.0, h, 0.01 * h)
    o_ref[...] = jnp.dot(a, w2_ref[...],
                         preferred_element_type=jnp.float32) * dinv


_mid = pl.pallas_call(
    _mid_body,
    grid=(NB,),
    in_specs=[
        pl.BlockSpec((2, RB, HP), lambda i: (0, i, 0)),
        pl.BlockSpec((RB, HP), lambda i: (i, 0)),
        pl.BlockSpec((2, 1, RB, 1), lambda i: (0, i, 0, 0)),
        pl.BlockSpec((1, HP), lambda i: (0, 0)),
        pl.BlockSpec((HP, HP), lambda i: (0, 0)),
    ],
    out_specs=pl.BlockSpec((RB, HP), lambda i: (i, 0)),
    out_shape=jax.ShapeDtypeStruct((NR, HP), jnp.float32),
)


def _pool_body(s_ref, p_ref, deg_ref, b2_ref, batch_ref, o_ref):
    i = pl.program_id(0)

    @pl.when(i == 0)
    def _():
        o_ref[...] = jnp.zeros_like(o_ref)

    d = deg_ref[0, 0] + deg_ref[1, 0] + 1.0
    dinv = lax.rsqrt(d)
    h = (s_ref[0] + s_ref[1] + p_ref[...]) * dinv + b2_ref[...]
    a = jnp.where(h >= 0.0, h, 0.01 * h)
    seg = batch_ref[0]
    ids = lax.broadcasted_iota(jnp.int32, (G, RB), 0)
    onehot = (seg == ids).astype(jnp.float32)
    o_ref[...] += jnp.dot(onehot, a, preferred_element_type=jnp.float32)


_pool = pl.pallas_call(
    _pool_body,
    grid=(NB,),
    in_specs=[
        pl.BlockSpec((2, RB, HP), lambda i: (0, i, 0)),
        pl.BlockSpec((RB, HP), lambda i: (i, 0)),
        pl.BlockSpec((2, 1, RB, 1), lambda i: (0, i, 0, 0)),
        pl.BlockSpec((1, HP), lambda i: (0, 0)),
        pl.BlockSpec((1, 1, RB), lambda i: (i, 0, 0)),
    ],
    out_specs=pl.BlockSpec((G, HP), lambda i: (0, 0)),
    out_shape=jax.ShapeDtypeStruct((G, HP), jnp.float32),
)


def _head_body(pool_ref, ws_ref, w0_ref, b0_ref, o_ref):
    o_ref[...] = (jnp.dot(pool_ref[...], ws_ref[...],
                          preferred_element_type=jnp.float32)
                  + w0_ref[...] + b0_ref[...])


_head = pl.pallas_call(
    _head_body,
    grid=(1,),
    in_specs=[
        pl.BlockSpec((G, HP), lambda i: (0, 0)),
        pl.BlockSpec((HP, OUT), lambda i: (0, 0)),
        pl.BlockSpec((1, OUT), lambda i: (0, 0)),
        pl.BlockSpec((1, OUT), lambda i: (0, 0)),
    ],
    out_specs=pl.BlockSpec((G, OUT), lambda i: (0, 0)),
    out_shape=jax.ShapeDtypeStruct((G, OUT), jnp.float32),
)


def kernel(x, edge_index, batch, features,
           W1, b1, W2, b2, Wa, ba, Wb, bb, Wc, bc, Wo, bo):
    del features, Wa, ba, Wb, bb, Wc, bc
    padv = (jnp.arange(EP - E, dtype=jnp.int32) % (NR - N)) + N
    src2 = jnp.concatenate([edge_index[0], padv]).reshape(EP // 128, 128)
    dst2 = jnp.concatenate([edge_index[1], padv]).reshape(EP // 128, 128)
    W1p = jnp.pad(W1, ((0, 0), (0, HP - HID)))
    W2p = jnp.pad(W2, ((0, HP - HID), (0, HP - HID)))
    b1p = jnp.pad(b1, (0, HP - HID)).reshape(1, HP)
    b2p = jnp.pad(b2, (0, HP - HID)).reshape(1, HP)
    Wsp = jnp.pad(Wo[1:, :], ((0, HP - HID), (0, 0)))
    w0 = Wo[0, :].reshape(1, OUT)
    b0 = bo.reshape(1, OUT)
    batch3 = batch.reshape(NB, 1, RB)
    zer = jnp.zeros((RPT, HP), jnp.float32)

    degp = _deg_kernel(dst2)
    deg4 = degp[:, :N].reshape(2, NB, RB, 1)
    p1 = _mm1(x, deg4, W1p)
    s1 = _prop_kernel(src2, dst2, p1, zer)
    p2 = _mid(s1, p1, deg4, b1p, W2p)
    s2 = _prop_kernel(src2, dst2, p2, zer)
    pooled = _pool(s2, p2, deg4, b2p, batch3)
    return _head(pooled, Wsp, w0, b0)


# speedup vs baseline: 1.4468x; 1.1821x over previous
import functools

import jax
import jax.numpy as jnp
from jax import lax
from jax.experimental import pallas as pl
from jax.experimental.pallas import tpu as pltpu
from jax.experimental.pallas import tpu_sc as plsc

N = 100000
E = 3200000
G = 64
INCH = 128
HID = 12
HP = 16
OUT = 16
NRD = 102400
NR = 100016
CH = 1024
RPC = CH // 128
NW = 32
NCH = -(-E // (NW * CH))
EP = NW * NCH * CH
RPTD = NRD // 16
RPT = NR // 16
RB = 5000
NB = N // RB

_mesh = plsc.VectorSubcoreMesh(core_axis_name="c", subcore_axis_name="s")


@functools.partial(
    pl.kernel,
    out_type=jax.ShapeDtypeStruct((2, NRD), jnp.float32),
    mesh=_mesh,
    scratch_types=[
        pltpu.VMEM((128,), jnp.float32),
        pltpu.VMEM((1024,), jnp.float32),
        pltpu.VMEM((RPC, 128), jnp.int32),
        pltpu.VMEM_SHARED((NRD,), jnp.float32),
        pltpu.SemaphoreType.DMA,
    ],
    compiler_params=pltpu.CompilerParams(use_tc_tiling_on_sc=False),
)
def _deg_kernel(dst_hbm, deg_out, ones_v, zeros_v, idx_v, acc, sem):
    cid = lax.axis_index("c")
    sid = lax.axis_index("s")
    wid = cid * 16 + sid
    for k in range(8):
        ones_v[pl.ds(k * 16, 16)] = jnp.ones((16,), jnp.float32)

    def _z(i, c):
        zeros_v[pl.ds(i * 16, 16)] = jnp.zeros((16,), jnp.float32)
        return c

    lax.fori_loop(0, 64, _z, 0)
    base = sid * RPTD
    for k in range(6):
        pltpu.sync_copy(zeros_v, acc.at[pl.ds(base + k * 1024, 1024)])
    pltpu.sync_copy(zeros_v.at[pl.ds(0, 256)], acc.at[pl.ds(base + 6144, 256)])
    plsc.subcore_barrier()

    def _chunk(c, carry):
        row0 = (wid * NCH + c) * RPC
        pltpu.sync_copy(dst_hbm.at[pl.ds(row0, RPC)], idx_v)
        cps = [
            pltpu.async_copy(ones_v, acc.at[idx_v.at[j]], sem, add=True)
            for j in range(RPC)
        ]
        for cp in cps:
            cp.wait()
        return carry

    lax.fori_loop(0, NCH, _chunk, 0)
    plsc.subcore_barrier()
    pltpu.sync_copy(acc.at[pl.ds(base, RPTD)],
                    deg_out.at[cid, pl.ds(base, RPTD)])


@functools.partial(
    pl.kernel,
    out_type=jax.ShapeDtypeStruct((2, NR, HP), jnp.float32),
    mesh=_mesh,
    scratch_types=[
        pltpu.VMEM((RPC, 128), jnp.int32),
        pltpu.VMEM((RPC, 128), jnp.int32),
        pltpu.VMEM((CH, HP), jnp.float32),
        pltpu.VMEM_SHARED((NR, HP), jnp.float32),
        pltpu.SemaphoreType.DMA,
        pltpu.SemaphoreType.DMA,
    ],
    compiler_params=pltpu.CompilerParams(use_tc_tiling_on_sc=False),
)
def _prop_kernel(src_hbm, dst_hbm, tab_hbm, zer_hbm, out_hbm,
                 isrc, idst, rows, acc, gsem, ssem):
    cid = lax.axis_index("c")
    sid = lax.axis_index("s")
    wid = cid * 16 + sid
    base = sid * RPT
    pltpu.sync_copy(zer_hbm, acc.at[pl.ds(base, RPT)])
    plsc.subcore_barrier()

    dummy = jnp.broadcast_to(N + (wid % (NR - N)), (16,)).astype(jnp.int32)
    for k in range(8 * RPC):
        idst[k // 8, pl.ds((k % 8) * 16, 16)] = dummy
    for j in range(RPC):
        pltpu.async_copy(rows.at[pl.ds(j * 128, 128)],
                         acc.at[idst.at[j]], ssem, add=True)

    def _chunk(c, carry):
        pltpu.make_async_copy(zer_hbm.at[pl.ds(0, CH)], rows, ssem).wait()
        row0 = (wid * NCH + c) * RPC
        pltpu.sync_copy(src_hbm.at[pl.ds(row0, RPC)], isrc)
        pltpu.sync_copy(dst_hbm.at[pl.ds(row0, RPC)], idst)
        gs = [
            pltpu.async_copy(tab_hbm.at[isrc.at[j]],
                             rows.at[pl.ds(j * 128, 128)], gsem)
            for j in range(RPC)
        ]
        for j in range(RPC):
            gs[j].wait()
            pltpu.async_copy(rows.at[pl.ds(j * 128, 128)],
                             acc.at[idst.at[j]], ssem, add=True)
        return carry

    lax.fori_loop(0, NCH, _chunk, 0)
    pltpu.make_async_copy(zer_hbm.at[pl.ds(0, CH)], rows, ssem).wait()
    plsc.subcore_barrier()
    pltpu.sync_copy(acc.at[pl.ds(base, RPT)],
                    out_hbm.at[cid, pl.ds(base, RPT)])


def _mm1_body(x_ref, deg_ref, w_ref, p_ref):
    d = deg_ref[0, 0, 0, :] + deg_ref[1, 0, 0, :] + 1.0
    dinv = lax.rsqrt(d)[:, None]
    h = jnp.dot(x_ref[...], w_ref[...], preferred_element_type=jnp.float32)
    p_ref[...] = h * dinv


_mm1 = pl.pallas_call(
    _mm1_body,
    grid=(NB,),
    in_specs=[
        pl.BlockSpec((RB, INCH), lambda i: (i, 0)),
        pl.BlockSpec((2, 1, 1, RB), lambda i: (0, i, 0, 0)),
        pl.BlockSpec((INCH, HP), lambda i: (0, 0)),
    ],
    out_specs=pl.BlockSpec((RB, HP), lambda i: (i, 0)),
    out_shape=jax.ShapeDtypeStruct((NR, HP), jnp.float32),
)


def _mid_body(s_ref, p_ref, deg_ref, b1_ref, w2_ref, o_ref):
    d = deg_ref[0, 0, 0, :] + deg_ref[1, 0, 0, :] + 1.0
    dinv = lax.rsqrt(d)[:, None]
    h = (s_ref[0] + s_ref[1] + p_ref[...]) * dinv + b1_ref[...]
    a = jnp.where(h >= 0.0, h, 0.01 * h)
    o_ref[...] = jnp.dot(a, w2_ref[...],
                         preferred_element_type=jnp.float32) * dinv


_mid = pl.pallas_call(
    _mid_body,
    grid=(NB,),
    in_specs=[
        pl.BlockSpec((2, RB, HP), lambda i: (0, i, 0)),
        pl.BlockSpec((RB, HP), lambda i: (i, 0)),
        pl.BlockSpec((2, 1, 1, RB), lambda i: (0, i, 0, 0)),
        pl.BlockSpec((1, HP), lambda i: (0, 0)),
        pl.BlockSpec((HP, HP), lambda i: (0, 0)),
    ],
    out_specs=pl.BlockSpec((RB, HP), lambda i: (i, 0)),
    out_shape=jax.ShapeDtypeStruct((NR, HP), jnp.float32),
)


def _pool_body(s_ref, p_ref, deg_ref, b2_ref, batch_ref, o_ref):
    i = pl.program_id(0)

    @pl.when(i == 0)
    def _():
        o_ref[...] = jnp.zeros_like(o_ref)

    d = deg_ref[0, 0, 0, :] + deg_ref[1, 0, 0, :] + 1.0
    dinv = lax.rsqrt(d)[:, None]
    h = (s_ref[0] + s_ref[1] + p_ref[...]) * dinv + b2_ref[...]
    a = jnp.where(h >= 0.0, h, 0.01 * h)
    seg = batch_ref[0]
    ids = lax.broadcasted_iota(jnp.int32, (G, RB), 0)
    onehot = (seg == ids).astype(jnp.float32)
    o_ref[...] += jnp.dot(onehot, a, preferred_element_type=jnp.float32)


_pool = pl.pallas_call(
    _pool_body,
    grid=(NB,),
    in_specs=[
        pl.BlockSpec((2, RB, HP), lambda i: (0, i, 0)),
        pl.BlockSpec((RB, HP), lambda i: (i, 0)),
        pl.BlockSpec((2, 1, 1, RB), lambda i: (0, i, 0, 0)),
        pl.BlockSpec((1, HP), lambda i: (0, 0)),
        pl.BlockSpec((1, 1, RB), lambda i: (i, 0, 0)),
    ],
    out_specs=pl.BlockSpec((G, HP), lambda i: (0, 0)),
    out_shape=jax.ShapeDtypeStruct((G, HP), jnp.float32),
)


def _head_body(pool_ref, ws_ref, w0_ref, b0_ref, o_ref):
    o_ref[...] = (jnp.dot(pool_ref[...], ws_ref[...],
                          preferred_element_type=jnp.float32)
                  + w0_ref[...] + b0_ref[...])


_head = pl.pallas_call(
    _head_body,
    grid=(1,),
    in_specs=[
        pl.BlockSpec((G, HP), lambda i: (0, 0)),
        pl.BlockSpec((HP, OUT), lambda i: (0, 0)),
        pl.BlockSpec((1, OUT), lambda i: (0, 0)),
        pl.BlockSpec((1, OUT), lambda i: (0, 0)),
    ],
    out_specs=pl.BlockSpec((G, OUT), lambda i: (0, 0)),
    out_shape=jax.ShapeDtypeStruct((G, OUT), jnp.float32),
)


def kernel(x, edge_index, batch, features,
           W1, b1, W2, b2, Wa, ba, Wb, bb, Wc, bc, Wo, bo):
    del features, Wa, ba, Wb, bb, Wc, bc
    padv = (jnp.arange(EP - E, dtype=jnp.int32) % (NR - N)) + N
    src2 = jnp.concatenate([edge_index[0], padv]).reshape(EP // 128, 128)
    dst2 = jnp.concatenate([edge_index[1], padv]).reshape(EP // 128, 128)
    W1p = jnp.pad(W1, ((0, 0), (0, HP - HID)))
    W2p = jnp.pad(W2, ((0, HP - HID), (0, HP - HID)))
    b1p = jnp.pad(b1, (0, HP - HID)).reshape(1, HP)
    b2p = jnp.pad(b2, (0, HP - HID)).reshape(1, HP)
    Wsp = jnp.pad(Wo[1:, :], ((0, HP - HID), (0, 0)))
    w0 = Wo[0, :].reshape(1, OUT)
    b0 = bo.reshape(1, OUT)
    batch3 = batch.reshape(NB, 1, RB)
    zer = jnp.zeros((RPT, HP), jnp.float32)

    degp = _deg_kernel(dst2)
    deg4 = degp[:, :N].reshape(2, NB, 1, RB)
    p1 = _mm1(x, deg4, W1p)
    s1 = _prop_kernel(src2, dst2, p1, zer)
    p2 = _mid(s1, p1, deg4, b1p, W2p)
    s2 = _prop_kernel(src2, dst2, p2, zer)
    pooled = _pool(s2, p2, deg4, b2p, batch3)
    return _head(pooled, Wsp, w0, b0)
